# baseline (speedup 1.0000x reference)
import functools

import jax
import jax.numpy as jnp
from jax import lax
from jax.experimental import pallas as pl
from jax.experimental.pallas import tpu as pltpu
from jax.experimental.pallas import tpu_sc as plsc

N = 100000
E = 3200000
G = 1024
NC = 2
NS = 16
NW = NC * NS
EG = E // 128
NP = 100096
ROWS_PER_TILE = NP // NS
ZR = 368
RPT = NP // NW
RCH = 136

_mesh = plsc.VectorSubcoreMesh(core_axis_name="c", subcore_axis_name="s")
_sc_params = pltpu.CompilerParams(needs_layout_passes=False,
                                  use_tc_tiling_on_sc=False)
f32 = jnp.float32
i32 = jnp.int32


@functools.partial(
    pl.kernel,
    out_type=jax.ShapeDtypeStruct((NW, NP), f32),
    mesh=_mesh,
    compiler_params=_sc_params,
    scratch_types=[
        pltpu.VMEM((NP,), f32),
        pltpu.VMEM((2, 16, 128), i32),
        pltpu.VMEM((2, 16, 128), f32),
        pltpu.SemaphoreType.DMA,
        pltpu.SemaphoreType.DMA,
    ],
)
def _deg(col2d, ew2d, out_hbm, acc, colb, ewb, dm0, dm1):
    c = lax.axis_index("c")
    s = lax.axis_index("s")
    wid = s * NC + c

    def zero(i, _):
        acc[pl.ds(i * 16, 16)] = jnp.zeros((16,), f32)
        return 0

    lax.fori_loop(0, NP // 16, zero, 0)

    g0 = (wid * EG) // NW
    g1 = ((wid + 1) * EG) // NW
    nb = (g1 - g0) // 16

    def issue(cix, buf):
        base = g0 + cix * 16
        pltpu.async_copy(col2d.at[pl.ds(base, 16), :], colb.at[buf], dm0)
        pltpu.async_copy(ew2d.at[pl.ds(base, 16), :], ewb.at[buf], dm1)

    def wait(buf):
        pltpu.make_async_copy(col2d.at[pl.ds(0, 16), :], colb.at[buf],
                              dm0).wait()
        pltpu.make_async_copy(ew2d.at[pl.ds(0, 16), :], ewb.at[buf],
                              dm1).wait()

    def process(buf):
        for j in range(16):
            for k in range(8):
                idx = colb[buf, j, pl.ds(k * 16, 16)]
                w = ewb[buf, j, pl.ds(k * 16, 16)]
                plsc.addupdate_scatter(acc, [idx], w)

    issue(0, 0)

    def chunk(cix, _):
        @pl.when(cix + 1 < nb)
        def _pre():
            issue(cix + 1, (cix + 1) % 2)

        wait(cix % 2)
        process(cix % 2)
        return 0

    lax.fori_loop(0, nb, chunk, 0)

    def tail(g, _):
        pltpu.sync_copy(col2d.at[g], colb.at[0, 0])
        pltpu.sync_copy(ew2d.at[g], ewb.at[0, 0])
        for k in range(8):
            idx = colb[0, 0, pl.ds(k * 16, 16)]
            w = ewb[0, 0, pl.ds(k * 16, 16)]
            plsc.addupdate_scatter(acc, [idx], w)
        return 0

    lax.fori_loop(g0 + nb * 16, g1, tail, 0)
    pltpu.sync_copy(acc, out_hbm.at[wid])


def _make_repack(nario):
    @functools.partial(
        pl.kernel,
        out_type=tuple(jax.ShapeDtypeStruct((NP, 16), f32)
                       for _ in range(nario)),
        mesh=_mesh,
        compiler_params=_sc_params,
        scratch_types=[pltpu.VMEM((RCH, 16), f32)] * nario
        + [pltpu.SemaphoreType.DMA] * nario,
    )
    def _repack(*args):
        srcs = args[:nario]
        outs = args[nario:2 * nario]
        bufs = args[2 * nario:3 * nario]
        sems = args[3 * nario:]
        c = lax.axis_index("c")
        s = lax.axis_index("s")
        wid = s * NC + c

        def body(i, _):
            r = wid * RPT + i * RCH
            for t in range(nario):
                pltpu.async_copy(srcs[t].at[pl.ds(r, RCH), :], bufs[t],
                                 sems[t])
            for t in range(nario):
                pltpu.make_async_copy(srcs[t].at[pl.ds(r, RCH), :], bufs[t],
                                      sems[t]).wait()
                pltpu.sync_copy(bufs[t], outs[t].at[pl.ds(r, RCH), :])
            return 0

        lax.fori_loop(0, RPT // RCH, body, 0)

    return _repack


_repack_one = _make_repack(1)
_repack_two = _make_repack(2)


def _make_prop(split_edges):
    @functools.partial(
        pl.kernel,
        out_type=jax.ShapeDtypeStruct((NC, NP, 16), f32),
        mesh=_mesh,
        compiler_params=_sc_params,
        scratch_types=[
            pltpu.VMEM_SHARED((NP, 16), f32),
            pltpu.VMEM((16, 128), i32),
            pltpu.VMEM((16, 128), i32),
            pltpu.VMEM((16, 128), f32),
            pltpu.VMEM((128, 16), f32),
            pltpu.VMEM((128, 16), f32),
            pltpu.VMEM((128, 16), f32),
            pltpu.VMEM((128, 16), f32),
            pltpu.VMEM((128, 16), f32),
            pltpu.VMEM((128, 16), f32),
            pltpu.VMEM((128, 16), f32),
            pltpu.VMEM((128, 16), f32),
            pltpu.VMEM((ZR, 16), f32),
        ] + [pltpu.SemaphoreType.DMA] * 16,
    )
    def _prop(src0, src1, row2d, col2d, ew2d, out_hbm,
              acc, rowb, colb, ewb,
              msg0, msg1, msg2, msg3, msg4, msg5, msg6, msg7, stage,
              gs0, gs1, gs2, gs3, gs4, gs5, gs6, gs7,
              ss0, ss1, ss2, ss3, ss4, ss5, ss6, ss7):
        c = lax.axis_index("c")
        s = lax.axis_index("s")
        msgs = (msg0, msg1, msg2, msg3, msg4, msg5, msg6, msg7)
        gsems = (gs0, gs1, gs2, gs3, gs4, gs5, gs6, gs7)
        ssems = (ss0, ss1, ss2, ss3, ss4, ss5, ss6, ss7)

        def zstage(i, _):
            stage[i, :] = jnp.zeros((16,), f32)
            return 0

        lax.fori_loop(0, ZR, zstage, 0)
        base_rows = s * ROWS_PER_TILE

        def zacc(i, _):
            pltpu.sync_copy(stage, acc.at[pl.ds(base_rows + i * ZR, ZR), :])
            return 0

        lax.fori_loop(0, ROWS_PER_TILE // ZR, zacc, 0)
        plsc.subcore_barrier()

        if split_edges:
            half = EG // NC
            g0 = c * half + (s * half) // NS
            g1 = c * half + ((s + 1) * half) // NS
        else:
            g0 = (s * EG) // NS
            g1 = ((s + 1) * EG) // NS
        nb = (g1 - g0) // 16

        def gather(idx_slice, msg, sem):
            @pl.when(c == 0)
            def _g0():
                pltpu.async_copy(src0.at[idx_slice], msg, sem)

            @pl.when(c == 1)
            def _g1():
                pltpu.async_copy(src1.at[idx_slice], msg, sem)

        def drain_gather(msg, sem):
            @pl.when(c == 0)
            def _d0():
                pltpu.make_async_copy(src0.at[rowb.at[0]], msg, sem).wait()

            @pl.when(c == 1)
            def _d1():
                pltpu.make_async_copy(src1.at[rowb.at[0]], msg, sem).wait()

        def drain_scatter(msg, sem):
            pltpu.make_async_copy(msg, acc.at[colb.at[0]], sem).wait()

        def scale(msg, j):
            for q in range(8):
                wv = ewb[j, pl.ds(q * 16, 16)]
                for t in range(16):
                    k = q * 16 + t
                    msg[k, :] = msg[k, :] * wv[t]

        def chunk(cix, _):
            base = g0 + cix * 16
            pltpu.sync_copy(row2d.at[pl.ds(base, 16), :], rowb)
            pltpu.sync_copy(col2d.at[pl.ds(base, 16), :], colb)
            pltpu.sync_copy(ew2d.at[pl.ds(base, 16), :], ewb)

            def inner(jj, _):
                j0 = jj * 8
                for b in range(8):
                    @pl.when(cix * 2 + jj > 0)
                    def _ds(b=b):
                        drain_scatter(msgs[b], ssems[b])

                    gather(rowb.at[j0 + b], msgs[b], gsems[b])
                for b in range(8):
                    drain_gather(msgs[b], gsems[b])
                    scale(msgs[b], j0 + b)
                    pltpu.async_copy(msgs[b], acc.at[colb.at[j0 + b]],
                                     ssems[b], add=True)
                return 0

            lax.fori_loop(0, 2, inner, 0)
            return 0

        lax.fori_loop(0, nb, chunk, 0)
        for b in range(8):
            drain_scatter(msgs[b], ssems[b])

        def tail(g, _):
            pltpu.sync_copy(row2d.at[g], rowb.at[0])
            pltpu.sync_copy(col2d.at[g], colb.at[0])
            pltpu.sync_copy(ew2d.at[g], ewb.at[0])
            gather(rowb.at[0], msg0, gs0)
            drain_gather(msg0, gs0)
            scale(msg0, 0)
            pltpu.sync_copy(msg0, acc.at[colb.at[0]], add=True)
            return 0

        lax.fori_loop(g0 + nb * 16, g1, tail, 0)
        plsc.subcore_barrier()

        def wb(i, _):
            r = base_rows + i * ZR
            pltpu.sync_copy(acc.at[pl.ds(r, ZR), :], stage)
            pltpu.sync_copy(stage, out_hbm.at[c, pl.ds(r, ZR), :])
            return 0

        lax.fori_loop(0, ROWS_PER_TILE // ZR, wb, 0)

    return _prop


_prop_l1 = _make_prop(split_edges=True)
_prop_l2 = _make_prop(split_edges=False)


_PC = 80
_PG = N // _PC


@functools.partial(
    pl.kernel,
    out_type=(
        jax.ShapeDtypeStruct((NC, G, 32), f32),
        jax.ShapeDtypeStruct((NW, G), f32),
    ),
    mesh=_mesh,
    compiler_params=_sc_params,
    scratch_types=[
        pltpu.VMEM_SHARED((G, 32), f32),
        pltpu.VMEM((G,), f32),
        pltpu.VMEM((_PC, 32), f32),
        pltpu.VMEM((_PC,), i32),
        pltpu.VMEM((64, 32), f32),
    ],
)
def _pool(h2, batch2d, sums_hbm, cnt_hbm, acc, cnt, hb, bb, stage):
    c = lax.axis_index("c")
    s = lax.axis_index("s")
    wid = s * NC + c

    def zst(i, _):
        stage[i, pl.ds(0, 16)] = jnp.zeros((16,), f32)
        stage[i, pl.ds(16, 16)] = jnp.zeros((16,), f32)
        return 0

    lax.fori_loop(0, 64, zst, 0)
    pltpu.sync_copy(stage, acc.at[pl.ds(s * 64, 64), :])

    def zc(i, _):
        cnt[pl.ds(i * 16, 16)] = jnp.zeros((16,), f32)
        return 0

    lax.fori_loop(0, G // 16, zc, 0)
    plsc.subcore_barrier()

    ones = jnp.ones((16,), f32)

    def body(g, _):
        r = g * _PC
        pltpu.sync_copy(h2.at[pl.ds(r, _PC), :], hb)
        pltpu.sync_copy(batch2d.at[g], bb)
        for k in range(_PC // 16):
            idx = bb[pl.ds(k * 16, 16)]
            plsc.addupdate_scatter(cnt, [idx], ones)
        pltpu.sync_copy(hb, acc.at[bb], add=True)
        return 0

    lax.fori_loop((wid * _PG) // NW, ((wid + 1) * _PG) // NW, body, 0)
    plsc.subcore_barrier()

    pltpu.sync_copy(acc.at[pl.ds(s * 64, 64), :], stage)
    pltpu.sync_copy(stage, sums_hbm.at[c, pl.ds(s * 64, 64), :])
    pltpu.sync_copy(cnt, cnt_hbm.at[wid])


_BN = 4352
_GRID = NP // _BN


def _tc_xs_body(degp_ref, x_ref, xs_ref):
    off = pl.multiple_of(pl.program_id(0) * _BN, 256)
    deg = 1.0 + jnp.sum(degp_ref[:, pl.ds(off, _BN)], axis=0)
    dinv = jnp.where(deg > 0, lax.rsqrt(jnp.where(deg > 0, deg, 1.0)), 0.0)
    dinv = dinv[:, None]
    xs4 = x_ref[...] * dinv
    xs_ref[...] = jnp.concatenate(
        [xs4, dinv, jnp.zeros((_BN, 11), f32)], axis=1)


def _tc_xs(degp, x):
    return pl.pallas_call(
        _tc_xs_body,
        grid=(_GRID,),
        in_specs=[
            pl.BlockSpec((NW, NP), lambda i: (0, 0)),
            pl.BlockSpec((_BN, 4), lambda i: (i, 0)),
        ],
        out_specs=pl.BlockSpec((_BN, 16), lambda i: (i, 0)),
        out_shape=jax.ShapeDtypeStruct((NP, 16), f32),
    )(degp, x)


def _tc_layer1_body(s1p_ref, xsp_ref, w1_ref, b1_ref, hsa_ref, hsb_ref):
    S = s1p_ref[0] + s1p_ref[1]
    xsp = xsp_ref[...]
    dinv = xsp[:, 4:5]
    a1 = dinv * (S[:, :4] + xsp[:, :4])
    h1 = lax.dot_general(a1, w1_ref[...], (((1,), (0,)), ((), ())),
                         preferred_element_type=f32)
    h1 = jnp.maximum(h1 + b1_ref[...], 0.0)
    hs = dinv * h1
    hsa_ref[...] = hs[:, :16]
    hsb_ref[...] = hs[:, 16:]


def _tc_layer1(s1p, xsp, W1, b1):
    return pl.pallas_call(
        _tc_layer1_body,
        grid=(_GRID,),
        in_specs=[
            pl.BlockSpec((NC, _BN, 16), lambda i: (0, i, 0)),
            pl.BlockSpec((_BN, 16), lambda i: (i, 0)),
            pl.BlockSpec((4, 32), lambda i: (0, 0)),
            pl.BlockSpec((1, 32), lambda i: (0, 0)),
        ],
        out_specs=[
            pl.BlockSpec((_BN, 16), lambda i: (i, 0)),
            pl.BlockSpec((_BN, 16), lambda i: (i, 0)),
        ],
        out_shape=[
            jax.ShapeDtypeStruct((NP, 16), f32),
            jax.ShapeDtypeStruct((NP, 16), f32),
        ],
    )(s1p, xsp, W1, b1)


def _tc_layer2_body(s2p_ref, hsa_ref, hsb_ref, xsp_ref, w2_ref, b2_ref,
                    h2_ref):
    S2 = jnp.concatenate([s2p_ref[0], s2p_ref[1]], axis=1)
    hs = jnp.concatenate([hsa_ref[...], hsb_ref[...]], axis=1)
    dinv = xsp_ref[...][:, 4:5]
    a2 = dinv * (S2 + hs)
    h2 = lax.dot_general(a2, w2_ref[...], (((1,), (0,)), ((), ())),
                         preferred_element_type=f32)
    h2_ref[...] = jnp.maximum(h2 + b2_ref[...], 0.0)


def _tc_layer2(s2p, hsa, hsb, xsp, W2, b2):
    return pl.pallas_call(
        _tc_layer2_body,
        grid=(_GRID,),
        in_specs=[
            pl.BlockSpec((NC, _BN, 16), lambda i: (0, i, 0)),
            pl.BlockSpec((_BN, 16), lambda i: (i, 0)),
            pl.BlockSpec((_BN, 16), lambda i: (i, 0)),
            pl.BlockSpec((_BN, 16), lambda i: (i, 0)),
            pl.BlockSpec((32, 32), lambda i: (0, 0)),
            pl.BlockSpec((1, 32), lambda i: (0, 0)),
        ],
        out_specs=pl.BlockSpec((_BN, 32), lambda i: (i, 0)),
        out_shape=jax.ShapeDtypeStruct((NP, 32), f32),
    )(s2p, hsa, hsb, xsp, W2, b2)


def _tc_heads_body(sums_ref, cnt_ref, wo_ref, bo_ref, wb_ref, bb_ref,
                   o_ref, b_ref):
    sums = sums_ref[0] + sums_ref[1]
    cnt = jnp.sum(cnt_ref[...], axis=0)[:, None]
    pooled = sums / jnp.maximum(cnt, 1.0)
    o = lax.dot_general(pooled, wo_ref[...], (((1,), (0,)), ((), ())),
                        preferred_element_type=f32)
    b = lax.dot_general(pooled, wb_ref[...], (((1,), (0,)), ((), ())),
                        preferred_element_type=f32)
    o_ref[...] = jax.nn.sigmoid(o + bo_ref[...])
    b_ref[...] = jax.nn.sigmoid(b + bb_ref[...])


def _tc_heads(sums, cnt, Wo, bo, Wb, bb):
    return pl.pallas_call(
        _tc_heads_body,
        out_shape=[
            jax.ShapeDtypeStruct((G, 1), f32),
            jax.ShapeDtypeStruct((G, 1), f32),
        ],
    )(sums, cnt, Wo, bo, Wb, bb)


def kernel(x, edge_index, edge_weight, batch, W1, b1, W2, b2, Wo, bo, Wb, bb):
    row2d = edge_index[0].reshape(EG, 128)
    col2d = edge_index[1].reshape(EG, 128)
    ew2d = edge_weight.reshape(EG, 128)
    batch2d = batch.reshape(_PG, _PC)

    degp = _deg(col2d, ew2d)
    xsp = _tc_xs(degp, x)
    (xsg,) = _repack_one(xsp)
    s1p = _prop_l1(xsg, xsg, row2d, col2d, ew2d)
    hsa, hsb = _tc_layer1(s1p, xsp, W1, b1.reshape(1, 32))
    hsag, hsbg = _repack_two(hsa, hsb)
    s2p = _prop_l2(hsag, hsbg, row2d, col2d, ew2d)
    h2 = _tc_layer2(s2p, hsa, hsb, xsp, W2, b2.reshape(1, 32))
    sums, cnt = _pool(h2, batch2d)
    orange, blue = _tc_heads(sums, cnt, Wo, bo.reshape(1, 1),
                             Wb, bb.reshape(1, 1))
    return (orange, blue)


# speedup vs baseline: 1.1206x; 1.1206x over previous
import functools

import jax
import jax.numpy as jnp
from jax import lax
from jax.experimental import pallas as pl
from jax.experimental.pallas import tpu as pltpu
from jax.experimental.pallas import tpu_sc as plsc

N = 100000
E = 3200000
G = 1024
NC = 2
NS = 16
NW = NC * NS
EG = E // 128
NP = 100096
ROWS_PER_TILE = NP // NS
ZR = 368
RPT = NP // NW
RCH = 136

_mesh = plsc.VectorSubcoreMesh(core_axis_name="c", subcore_axis_name="s")
_sc_params = pltpu.CompilerParams(needs_layout_passes=False,
                                  use_tc_tiling_on_sc=False)
f32 = jnp.float32
i32 = jnp.int32


@functools.partial(
    pl.kernel,
    out_type=jax.ShapeDtypeStruct((NW, NP), f32),
    mesh=_mesh,
    compiler_params=_sc_params,
    scratch_types=[
        pltpu.VMEM((NP,), f32),
        pltpu.VMEM((2, 16, 128), i32),
        pltpu.VMEM((2, 16, 128), f32),
        pltpu.SemaphoreType.DMA,
        pltpu.SemaphoreType.DMA,
    ],
)
def _deg(col2d, ew2d, out_hbm, acc, colb, ewb, dm0, dm1):
    c = lax.axis_index("c")
    s = lax.axis_index("s")
    wid = s * NC + c

    def zero(i, _):
        acc[pl.ds(i * 16, 16)] = jnp.zeros((16,), f32)
        return 0

    lax.fori_loop(0, NP // 16, zero, 0)

    g0 = (wid * EG) // NW
    g1 = ((wid + 1) * EG) // NW
    nb = (g1 - g0) // 16

    def issue(cix, buf):
        base = g0 + cix * 16
        pltpu.async_copy(col2d.at[pl.ds(base, 16), :], colb.at[buf], dm0)
        pltpu.async_copy(ew2d.at[pl.ds(base, 16), :], ewb.at[buf], dm1)

    def wait(buf):
        pltpu.make_async_copy(col2d.at[pl.ds(0, 16), :], colb.at[buf],
                              dm0).wait()
        pltpu.make_async_copy(ew2d.at[pl.ds(0, 16), :], ewb.at[buf],
                              dm1).wait()

    def process(buf):
        for j in range(16):
            for k in range(8):
                idx = colb[buf, j, pl.ds(k * 16, 16)]
                w = ewb[buf, j, pl.ds(k * 16, 16)]
                plsc.addupdate_scatter(acc, [idx], w)

    issue(0, 0)

    def chunk(cix, _):
        @pl.when(cix + 1 < nb)
        def _pre():
            issue(cix + 1, (cix + 1) % 2)

        wait(cix % 2)
        process(cix % 2)
        return 0

    lax.fori_loop(0, nb, chunk, 0)

    def tail(g, _):
        pltpu.sync_copy(col2d.at[g], colb.at[0, 0])
        pltpu.sync_copy(ew2d.at[g], ewb.at[0, 0])
        for k in range(8):
            idx = colb[0, 0, pl.ds(k * 16, 16)]
            w = ewb[0, 0, pl.ds(k * 16, 16)]
            plsc.addupdate_scatter(acc, [idx], w)
        return 0

    lax.fori_loop(g0 + nb * 16, g1, tail, 0)
    pltpu.sync_copy(acc, out_hbm.at[wid])


def _make_repack(nario):
    @functools.partial(
        pl.kernel,
        out_type=tuple(jax.ShapeDtypeStruct((NP, 16), f32)
                       for _ in range(nario)),
        mesh=_mesh,
        compiler_params=_sc_params,
        scratch_types=[pltpu.VMEM((RCH, 16), f32)] * nario
        + [pltpu.SemaphoreType.DMA] * nario,
    )
    def _repack(*args):
        srcs = args[:nario]
        outs = args[nario:2 * nario]
        bufs = args[2 * nario:3 * nario]
        sems = args[3 * nario:]
        c = lax.axis_index("c")
        s = lax.axis_index("s")
        wid = s * NC + c

        def body(i, _):
            r = wid * RPT + i * RCH
            for t in range(nario):
                pltpu.async_copy(srcs[t].at[pl.ds(r, RCH), :], bufs[t],
                                 sems[t])
            for t in range(nario):
                pltpu.make_async_copy(srcs[t].at[pl.ds(r, RCH), :], bufs[t],
                                      sems[t]).wait()
                pltpu.sync_copy(bufs[t], outs[t].at[pl.ds(r, RCH), :])
            return 0

        lax.fori_loop(0, RPT // RCH, body, 0)

    return _repack


_repack_one = _make_repack(1)
_repack_two = _make_repack(2)


def _make_prop(split_edges):
    @functools.partial(
        pl.kernel,
        out_type=jax.ShapeDtypeStruct((NC, NP, 16), f32),
        mesh=_mesh,
        compiler_params=_sc_params,
        scratch_types=[
            pltpu.VMEM_SHARED((NP, 16), f32),
            pltpu.VMEM((16, 128), i32),
            pltpu.VMEM((16, 128), i32),
            pltpu.VMEM((16, 128), f32),
            pltpu.VMEM((128, 16), f32),
            pltpu.VMEM((128, 16), f32),
            pltpu.VMEM((128, 16), f32),
            pltpu.VMEM((128, 16), f32),
            pltpu.VMEM((ZR, 16), f32),
        ] + [pltpu.SemaphoreType.DMA] * 8,
    )
    def _prop(src0, src1, row2d, col2d, ew2d, out_hbm,
              acc, rowb, colb, ewb,
              msg0, msg1, msg2, msg3, stage,
              gs0, gs1, gs2, gs3, ss0, ss1, ss2, ss3):
        c = lax.axis_index("c")
        s = lax.axis_index("s")
        msgs = (msg0, msg1, msg2, msg3)
        gsems = (gs0, gs1, gs2, gs3)
        ssems = (ss0, ss1, ss2, ss3)

        def zstage(i, _):
            stage[i, :] = jnp.zeros((16,), f32)
            return 0

        lax.fori_loop(0, ZR, zstage, 0)
        base_rows = s * ROWS_PER_TILE

        def zacc(i, _):
            pltpu.sync_copy(stage, acc.at[pl.ds(base_rows + i * ZR, ZR), :])
            return 0

        lax.fori_loop(0, ROWS_PER_TILE // ZR, zacc, 0)
        plsc.subcore_barrier()

        if split_edges:
            half = EG // NC
            g0 = c * half + (s * half) // NS
            g1 = c * half + ((s + 1) * half) // NS
        else:
            g0 = (s * EG) // NS
            g1 = ((s + 1) * EG) // NS
        nb = (g1 - g0) // 16

        def gather(idx_slice, msg, sem):
            @pl.when(c == 0)
            def _g0():
                pltpu.async_copy(src0.at[idx_slice], msg, sem)

            @pl.when(c == 1)
            def _g1():
                pltpu.async_copy(src1.at[idx_slice], msg, sem)

        def drain_gather(msg, sem):
            @pl.when(c == 0)
            def _d0():
                pltpu.make_async_copy(src0.at[rowb.at[0]], msg, sem).wait()

            @pl.when(c == 1)
            def _d1():
                pltpu.make_async_copy(src1.at[rowb.at[0]], msg, sem).wait()

        def drain_scatter(msg, sem):
            pltpu.make_async_copy(msg, acc.at[colb.at[0]], sem).wait()

        def scale(msg, j):
            for q in range(8):
                wv = ewb[j, pl.ds(q * 16, 16)]
                for t in range(16):
                    k = q * 16 + t
                    msg[k, :] = msg[k, :] * wv[t]

        def chunk(cix, _):
            base = g0 + cix * 16
            pltpu.sync_copy(row2d.at[pl.ds(base, 16), :], rowb)
            pltpu.sync_copy(col2d.at[pl.ds(base, 16), :], colb)
            pltpu.sync_copy(ew2d.at[pl.ds(base, 16), :], ewb)

            def inner(jj, _):
                j0 = jj * 4
                for b in range(4):
                    @pl.when(cix * 4 + jj > 0)
                    def _ds(b=b):
                        drain_scatter(msgs[b], ssems[b])

                    gather(rowb.at[j0 + b], msgs[b], gsems[b])
                for b in range(4):
                    drain_gather(msgs[b], gsems[b])
                    scale(msgs[b], j0 + b)
                    pltpu.async_copy(msgs[b], acc.at[colb.at[j0 + b]],
                                     ssems[b], add=True)
                return 0

            lax.fori_loop(0, 4, inner, 0)
            return 0

        lax.fori_loop(0, nb, chunk, 0)
        for b in range(4):
            drain_scatter(msgs[b], ssems[b])

        def tail(g, _):
            pltpu.sync_copy(row2d.at[g], rowb.at[0])
            pltpu.sync_copy(col2d.at[g], colb.at[0])
            pltpu.sync_copy(ew2d.at[g], ewb.at[0])
            gather(rowb.at[0], msg0, gs0)
            drain_gather(msg0, gs0)
            scale(msg0, 0)
            pltpu.sync_copy(msg0, acc.at[colb.at[0]], add=True)
            return 0

        lax.fori_loop(g0 + nb * 16, g1, tail, 0)
        plsc.subcore_barrier()

        def wb(i, _):
            r = base_rows + i * ZR
            pltpu.sync_copy(acc.at[pl.ds(r, ZR), :], stage)
            pltpu.sync_copy(stage, out_hbm.at[c, pl.ds(r, ZR), :])
            return 0

        lax.fori_loop(0, ROWS_PER_TILE // ZR, wb, 0)

    return _prop


_prop_l1 = _make_prop(split_edges=True)
_prop_l2 = _make_prop(split_edges=False)


_PC = 80
_PG = N // _PC


@functools.partial(
    pl.kernel,
    out_type=(
        jax.ShapeDtypeStruct((NC, G, 32), f32),
        jax.ShapeDtypeStruct((NW, G), f32),
    ),
    mesh=_mesh,
    compiler_params=_sc_params,
    scratch_types=[
        pltpu.VMEM_SHARED((G, 32), f32),
        pltpu.VMEM((G,), f32),
        pltpu.VMEM((_PC, 32), f32),
        pltpu.VMEM((_PC,), i32),
        pltpu.VMEM((64, 32), f32),
    ],
)
def _pool(h2, batch2d, sums_hbm, cnt_hbm, acc, cnt, hb, bb, stage):
    c = lax.axis_index("c")
    s = lax.axis_index("s")
    wid = s * NC + c

    def zst(i, _):
        stage[i, pl.ds(0, 16)] = jnp.zeros((16,), f32)
        stage[i, pl.ds(16, 16)] = jnp.zeros((16,), f32)
        return 0

    lax.fori_loop(0, 64, zst, 0)
    pltpu.sync_copy(stage, acc.at[pl.ds(s * 64, 64), :])

    def zc(i, _):
        cnt[pl.ds(i * 16, 16)] = jnp.zeros((16,), f32)
        return 0

    lax.fori_loop(0, G // 16, zc, 0)
    plsc.subcore_barrier()

    ones = jnp.ones((16,), f32)

    def body(g, _):
        r = g * _PC
        pltpu.sync_copy(h2.at[pl.ds(r, _PC), :], hb)
        pltpu.sync_copy(batch2d.at[g], bb)
        for k in range(_PC // 16):
            idx = bb[pl.ds(k * 16, 16)]
            plsc.addupdate_scatter(cnt, [idx], ones)
        pltpu.sync_copy(hb, acc.at[bb], add=True)
        return 0

    lax.fori_loop((wid * _PG) // NW, ((wid + 1) * _PG) // NW, body, 0)
    plsc.subcore_barrier()

    pltpu.sync_copy(acc.at[pl.ds(s * 64, 64), :], stage)
    pltpu.sync_copy(stage, sums_hbm.at[c, pl.ds(s * 64, 64), :])
    pltpu.sync_copy(cnt, cnt_hbm.at[wid])


_BN = 4352
_GRID = NP // _BN


def _tc_xs_body(degp_ref, x_ref, xs_ref):
    off = pl.multiple_of(pl.program_id(0) * _BN, 256)
    deg = 1.0 + jnp.sum(degp_ref[:, pl.ds(off, _BN)], axis=0)
    dinv = jnp.where(deg > 0, lax.rsqrt(jnp.where(deg > 0, deg, 1.0)), 0.0)
    dinv = dinv[:, None]
    xs4 = x_ref[...] * dinv
    xs_ref[...] = jnp.concatenate(
        [xs4, dinv, jnp.zeros((_BN, 11), f32)], axis=1)


def _tc_xs(degp, x):
    return pl.pallas_call(
        _tc_xs_body,
        grid=(_GRID,),
        in_specs=[
            pl.BlockSpec((NW, NP), lambda i: (0, 0)),
            pl.BlockSpec((_BN, 4), lambda i: (i, 0)),
        ],
        out_specs=pl.BlockSpec((_BN, 16), lambda i: (i, 0)),
        out_shape=jax.ShapeDtypeStruct((NP, 16), f32),
    )(degp, x)


def _tc_layer1_body(s1p_ref, xsp_ref, w1_ref, b1_ref, hsa_ref, hsb_ref):
    S = s1p_ref[0] + s1p_ref[1]
    xsp = xsp_ref[...]
    dinv = xsp[:, 4:5]
    a1 = dinv * (S[:, :4] + xsp[:, :4])
    h1 = lax.dot_general(a1, w1_ref[...], (((1,), (0,)), ((), ())),
                         preferred_element_type=f32)
    h1 = jnp.maximum(h1 + b1_ref[...], 0.0)
    hs = dinv * h1
    hsa_ref[...] = hs[:, :16]
    hsb_ref[...] = hs[:, 16:]


def _tc_layer1(s1p, xsp, W1, b1):
    return pl.pallas_call(
        _tc_layer1_body,
        grid=(_GRID,),
        in_specs=[
            pl.BlockSpec((NC, _BN, 16), lambda i: (0, i, 0)),
            pl.BlockSpec((_BN, 16), lambda i: (i, 0)),
            pl.BlockSpec((4, 32), lambda i: (0, 0)),
            pl.BlockSpec((1, 32), lambda i: (0, 0)),
        ],
        out_specs=[
            pl.BlockSpec((_BN, 16), lambda i: (i, 0)),
            pl.BlockSpec((_BN, 16), lambda i: (i, 0)),
        ],
        out_shape=[
            jax.ShapeDtypeStruct((NP, 16), f32),
            jax.ShapeDtypeStruct((NP, 16), f32),
        ],
    )(s1p, xsp, W1, b1)


def _tc_layer2_body(s2p_ref, hsa_ref, hsb_ref, xsp_ref, w2_ref, b2_ref,
                    h2_ref):
    S2 = jnp.concatenate([s2p_ref[0], s2p_ref[1]], axis=1)
    hs = jnp.concatenate([hsa_ref[...], hsb_ref[...]], axis=1)
    dinv = xsp_ref[...][:, 4:5]
    a2 = dinv * (S2 + hs)
    h2 = lax.dot_general(a2, w2_ref[...], (((1,), (0,)), ((), ())),
                         preferred_element_type=f32)
    h2_ref[...] = jnp.maximum(h2 + b2_ref[...], 0.0)


def _tc_layer2(s2p, hsa, hsb, xsp, W2, b2):
    return pl.pallas_call(
        _tc_layer2_body,
        grid=(_GRID,),
        in_specs=[
            pl.BlockSpec((NC, _BN, 16), lambda i: (0, i, 0)),
            pl.BlockSpec((_BN, 16), lambda i: (i, 0)),
            pl.BlockSpec((_BN, 16), lambda i: (i, 0)),
            pl.BlockSpec((_BN, 16), lambda i: (i, 0)),
            pl.BlockSpec((32, 32), lambda i: (0, 0)),
            pl.BlockSpec((1, 32), lambda i: (0, 0)),
        ],
        out_specs=pl.BlockSpec((_BN, 32), lambda i: (i, 0)),
        out_shape=jax.ShapeDtypeStruct((NP, 32), f32),
    )(s2p, hsa, hsb, xsp, W2, b2)


def _tc_heads_body(sums_ref, cnt_ref, wo_ref, bo_ref, wb_ref, bb_ref,
                   o_ref, b_ref):
    sums = sums_ref[0] + sums_ref[1]
    cnt = jnp.sum(cnt_ref[...], axis=0)[:, None]
    pooled = sums / jnp.maximum(cnt, 1.0)
    o = lax.dot_general(pooled, wo_ref[...], (((1,), (0,)), ((), ())),
                        preferred_element_type=f32)
    b = lax.dot_general(pooled, wb_ref[...], (((1,), (0,)), ((), ())),
                        preferred_element_type=f32)
    o_ref[...] = jax.nn.sigmoid(o + bo_ref[...])
    b_ref[...] = jax.nn.sigmoid(b + bb_ref[...])


def _tc_heads(sums, cnt, Wo, bo, Wb, bb):
    return pl.pallas_call(
        _tc_heads_body,
        out_shape=[
            jax.ShapeDtypeStruct((G, 1), f32),
            jax.ShapeDtypeStruct((G, 1), f32),
        ],
    )(sums, cnt, Wo, bo, Wb, bb)


def kernel(x, edge_index, edge_weight, batch, W1, b1, W2, b2, Wo, bo, Wb, bb):
    row2d = edge_index[0].reshape(EG, 128)
    col2d = edge_index[1].reshape(EG, 128)
    ew2d = edge_weight.reshape(EG, 128)
    batch2d = batch.reshape(_PG, _PC)

    degp = _deg(col2d, ew2d)
    xsp = _tc_xs(degp, x)
    (xsg,) = _repack_one(xsp)
    s1p = _prop_l1(xsg, xsg, row2d, col2d, ew2d)
    hsa, hsb = _tc_layer1(s1p, xsp, W1, b1.reshape(1, 32))
    hsag, hsbg = _repack_two(hsa, hsb)
    s2p = _prop_l2(hsag, hsbg, row2d, col2d, ew2d)
    h2 = _tc_layer2(s2p, hsa, hsb, xsp, W2, b2.reshape(1, 32))
    sums, cnt = _pool(h2, batch2d)
    orange, blue = _tc_heads(sums, cnt, Wo, bo.reshape(1, 1),
                             Wb, bb.reshape(1, 1))
    return (orange, blue)


# speedup vs baseline: 1.1758x; 1.0493x over previous
import functools

import jax
import jax.numpy as jnp
from jax import lax
from jax.experimental import pallas as pl
from jax.experimental.pallas import tpu as pltpu
from jax.experimental.pallas import tpu_sc as plsc

N = 100000
E = 3200000
G = 1024
NC = 2
NS = 16
NW = NC * NS
EG = E // 128
NP = 100096
ROWS_PER_TILE = NP // NS
ZR = 368
RPT = NP // NW
RCH = 136

_mesh = plsc.VectorSubcoreMesh(core_axis_name="c", subcore_axis_name="s")
_sc_params = pltpu.CompilerParams(needs_layout_passes=False,
                                  use_tc_tiling_on_sc=False)
f32 = jnp.float32
i32 = jnp.int32


@functools.partial(
    pl.kernel,
    out_type=jax.ShapeDtypeStruct((NW, NP), f32),
    mesh=_mesh,
    compiler_params=_sc_params,
    scratch_types=[
        pltpu.VMEM((NP,), f32),
        pltpu.VMEM((2, 16, 128), i32),
        pltpu.VMEM((2, 16, 128), f32),
        pltpu.SemaphoreType.DMA,
        pltpu.SemaphoreType.DMA,
    ],
)
def _deg(col2d, ew2d, out_hbm, acc, colb, ewb, dm0, dm1):
    c = lax.axis_index("c")
    s = lax.axis_index("s")
    wid = s * NC + c

    def zero(i, _):
        acc[pl.ds(i * 16, 16)] = jnp.zeros((16,), f32)
        return 0

    lax.fori_loop(0, NP // 16, zero, 0)

    g0 = (wid * EG) // NW
    g1 = ((wid + 1) * EG) // NW
    nb = (g1 - g0) // 16

    def issue(cix, buf):
        base = g0 + cix * 16
        pltpu.async_copy(col2d.at[pl.ds(base, 16), :], colb.at[buf], dm0)
        pltpu.async_copy(ew2d.at[pl.ds(base, 16), :], ewb.at[buf], dm1)

    def wait(buf):
        pltpu.make_async_copy(col2d.at[pl.ds(0, 16), :], colb.at[buf],
                              dm0).wait()
        pltpu.make_async_copy(ew2d.at[pl.ds(0, 16), :], ewb.at[buf],
                              dm1).wait()

    def process(buf):
        for j in range(16):
            for k in range(8):
                idx = colb[buf, j, pl.ds(k * 16, 16)]
                w = ewb[buf, j, pl.ds(k * 16, 16)]
                plsc.addupdate_scatter(acc, [idx], w)

    issue(0, 0)

    def chunk(cix, _):
        @pl.when(cix + 1 < nb)
        def _pre():
            issue(cix + 1, (cix + 1) % 2)

        wait(cix % 2)
        process(cix % 2)
        return 0

    lax.fori_loop(0, nb, chunk, 0)

    def tail(g, _):
        pltpu.sync_copy(col2d.at[g], colb.at[0, 0])
        pltpu.sync_copy(ew2d.at[g], ewb.at[0, 0])
        for k in range(8):
            idx = colb[0, 0, pl.ds(k * 16, 16)]
            w = ewb[0, 0, pl.ds(k * 16, 16)]
            plsc.addupdate_scatter(acc, [idx], w)
        return 0

    lax.fori_loop(g0 + nb * 16, g1, tail, 0)
    pltpu.sync_copy(acc, out_hbm.at[wid])


def _make_repack(nario):
    @functools.partial(
        pl.kernel,
        out_type=tuple(jax.ShapeDtypeStruct((NP, 16), f32)
                       for _ in range(nario)),
        mesh=_mesh,
        compiler_params=_sc_params,
        scratch_types=[pltpu.VMEM((RCH, 16), f32)] * nario
        + [pltpu.SemaphoreType.DMA] * nario,
    )
    def _repack(*args):
        srcs = args[:nario]
        outs = args[nario:2 * nario]
        bufs = args[2 * nario:3 * nario]
        sems = args[3 * nario:]
        c = lax.axis_index("c")
        s = lax.axis_index("s")
        wid = s * NC + c

        def body(i, _):
            r = wid * RPT + i * RCH
            for t in range(nario):
                pltpu.async_copy(srcs[t].at[pl.ds(r, RCH), :], bufs[t],
                                 sems[t])
            for t in range(nario):
                pltpu.make_async_copy(srcs[t].at[pl.ds(r, RCH), :], bufs[t],
                                      sems[t]).wait()
                pltpu.sync_copy(bufs[t], outs[t].at[pl.ds(r, RCH), :])
            return 0

        lax.fori_loop(0, RPT // RCH, body, 0)

    return _repack


_repack_one = _make_repack(1)
_repack_two = _make_repack(2)


def _make_prop(split_edges):
    @functools.partial(
        pl.kernel,
        out_type=jax.ShapeDtypeStruct((NC, NP, 16), f32),
        mesh=_mesh,
        compiler_params=_sc_params,
        scratch_types=[
            pltpu.VMEM_SHARED((NP, 16), f32),
            pltpu.VMEM((32, 128), i32),
            pltpu.VMEM((32, 128), i32),
            pltpu.VMEM((32, 128), f32),
            pltpu.VMEM((128, 16), f32),
            pltpu.VMEM((128, 16), f32),
            pltpu.VMEM((128, 16), f32),
            pltpu.VMEM((128, 16), f32),
            pltpu.VMEM((ZR, 16), f32),
        ] + [pltpu.SemaphoreType.DMA] * 8,
    )
    def _prop(src0, src1, row2d, col2d, ew2d, out_hbm,
              acc, rowb, colb, ewb,
              msg0, msg1, msg2, msg3, stage,
              gs0, gs1, gs2, gs3, ss0, ss1, ss2, ss3):
        c = lax.axis_index("c")
        s = lax.axis_index("s")
        msgs = (msg0, msg1, msg2, msg3)
        gsems = (gs0, gs1, gs2, gs3)
        ssems = (ss0, ss1, ss2, ss3)

        def zstage(i, _):
            stage[i, :] = jnp.zeros((16,), f32)
            return 0

        lax.fori_loop(0, ZR, zstage, 0)
        base_rows = s * ROWS_PER_TILE

        def zacc(i, _):
            pltpu.sync_copy(stage, acc.at[pl.ds(base_rows + i * ZR, ZR), :])
            return 0

        lax.fori_loop(0, ROWS_PER_TILE // ZR, zacc, 0)
        plsc.subcore_barrier()

        if split_edges:
            half = EG // NC
            g0 = c * half + (s * half) // NS
            g1 = c * half + ((s + 1) * half) // NS
        else:
            g0 = (s * EG) // NS
            g1 = ((s + 1) * EG) // NS
        nb = (g1 - g0) // 32

        def gather(idx_slice, msg, sem):
            @pl.when(c == 0)
            def _g0():
                pltpu.async_copy(src0.at[idx_slice], msg, sem)

            @pl.when(c == 1)
            def _g1():
                pltpu.async_copy(src1.at[idx_slice], msg, sem)

        def drain_gather(msg, sem):
            @pl.when(c == 0)
            def _d0():
                pltpu.make_async_copy(src0.at[rowb.at[0]], msg, sem).wait()

            @pl.when(c == 1)
            def _d1():
                pltpu.make_async_copy(src1.at[rowb.at[0]], msg, sem).wait()

        def drain_scatter(msg, sem):
            pltpu.make_async_copy(msg, acc.at[colb.at[0]], sem).wait()

        def scale(msg, j):
            for q in range(8):
                wv = ewb[j, pl.ds(q * 16, 16)]
                for t in range(16):
                    k = q * 16 + t
                    msg[k, :] = msg[k, :] * wv[t]

        def chunk(cix, _):
            base = g0 + cix * 32
            pltpu.sync_copy(row2d.at[pl.ds(base, 32), :], rowb)
            pltpu.sync_copy(col2d.at[pl.ds(base, 32), :], colb)
            pltpu.sync_copy(ew2d.at[pl.ds(base, 32), :], ewb)

            def inner(jj, _):
                j0 = jj * 4
                for b in range(4):
                    @pl.when(cix * 8 + jj > 0)
                    def _ds(b=b):
                        drain_scatter(msgs[b], ssems[b])

                    gather(rowb.at[j0 + b], msgs[b], gsems[b])
                for b in range(4):
                    drain_gather(msgs[b], gsems[b])
                    scale(msgs[b], j0 + b)
                    pltpu.async_copy(msgs[b], acc.at[colb.at[j0 + b]],
                                     ssems[b], add=True)
                return 0

            lax.fori_loop(0, 8, inner, 0)
            return 0

        lax.fori_loop(0, nb, chunk, 0)
        for b in range(4):
            drain_scatter(msgs[b], ssems[b])

        def tail(g, _):
            pltpu.sync_copy(row2d.at[g], rowb.at[0])
            pltpu.sync_copy(col2d.at[g], colb.at[0])
            pltpu.sync_copy(ew2d.at[g], ewb.at[0])
            gather(rowb.at[0], msg0, gs0)
            drain_gather(msg0, gs0)
            scale(msg0, 0)
            pltpu.sync_copy(msg0, acc.at[colb.at[0]], add=True)
            return 0

        lax.fori_loop(g0 + nb * 32, g1, tail, 0)
        plsc.subcore_barrier()

        def wb(i, _):
            r = base_rows + i * ZR
            pltpu.sync_copy(acc.at[pl.ds(r, ZR), :], stage)
            pltpu.sync_copy(stage, out_hbm.at[c, pl.ds(r, ZR), :])
            return 0

        lax.fori_loop(0, ROWS_PER_TILE // ZR, wb, 0)

    return _prop


_prop_l1 = _make_prop(split_edges=True)
_prop_l2 = _make_prop(split_edges=False)


_PC = 80
_PG = N // _PC


@functools.partial(
    pl.kernel,
    out_type=(
        jax.ShapeDtypeStruct((NC, G, 32), f32),
        jax.ShapeDtypeStruct((NW, G), f32),
    ),
    mesh=_mesh,
    compiler_params=_sc_params,
    scratch_types=[
        pltpu.VMEM_SHARED((G, 32), f32),
        pltpu.VMEM((G,), f32),
        pltpu.VMEM((_PC, 32), f32),
        pltpu.VMEM((_PC,), i32),
        pltpu.VMEM((64, 32), f32),
    ],
)
def _pool(h2, batch2d, sums_hbm, cnt_hbm, acc, cnt, hb, bb, stage):
    c = lax.axis_index("c")
    s = lax.axis_index("s")
    wid = s * NC + c

    def zst(i, _):
        stage[i, pl.ds(0, 16)] = jnp.zeros((16,), f32)
        stage[i, pl.ds(16, 16)] = jnp.zeros((16,), f32)
        return 0

    lax.fori_loop(0, 64, zst, 0)
    pltpu.sync_copy(stage, acc.at[pl.ds(s * 64, 64), :])

    def zc(i, _):
        cnt[pl.ds(i * 16, 16)] = jnp.zeros((16,), f32)
        return 0

    lax.fori_loop(0, G // 16, zc, 0)
    plsc.subcore_barrier()

    ones = jnp.ones((16,), f32)

    def body(g, _):
        r = g * _PC
        pltpu.sync_copy(h2.at[pl.ds(r, _PC), :], hb)
        pltpu.sync_copy(batch2d.at[g], bb)
        for k in range(_PC // 16):
            idx = bb[pl.ds(k * 16, 16)]
            plsc.addupdate_scatter(cnt, [idx], ones)
        pltpu.sync_copy(hb, acc.at[bb], add=True)
        return 0

    lax.fori_loop((wid * _PG) // NW, ((wid + 1) * _PG) // NW, body, 0)
    plsc.subcore_barrier()

    pltpu.sync_copy(acc.at[pl.ds(s * 64, 64), :], stage)
    pltpu.sync_copy(stage, sums_hbm.at[c, pl.ds(s * 64, 64), :])
    pltpu.sync_copy(cnt, cnt_hbm.at[wid])


_BN = 4352
_GRID = NP // _BN


def _tc_xs_body(degp_ref, x_ref, xs_ref):
    off = pl.multiple_of(pl.program_id(0) * _BN, 256)
    deg = 1.0 + jnp.sum(degp_ref[:, pl.ds(off, _BN)], axis=0)
    dinv = jnp.where(deg > 0, lax.rsqrt(jnp.where(deg > 0, deg, 1.0)), 0.0)
    dinv = dinv[:, None]
    xs4 = x_ref[...] * dinv
    xs_ref[...] = jnp.concatenate(
        [xs4, dinv, jnp.zeros((_BN, 11), f32)], axis=1)


def _tc_xs(degp, x):
    return pl.pallas_call(
        _tc_xs_body,
        grid=(_GRID,),
        in_specs=[
            pl.BlockSpec((NW, NP), lambda i: (0, 0)),
            pl.BlockSpec((_BN, 4), lambda i: (i, 0)),
        ],
        out_specs=pl.BlockSpec((_BN, 16), lambda i: (i, 0)),
        out_shape=jax.ShapeDtypeStruct((NP, 16), f32),
    )(degp, x)


def _tc_layer1_body(s1p_ref, xsp_ref, w1_ref, b1_ref, hsa_ref, hsb_ref):
    S = s1p_ref[0] + s1p_ref[1]
    xsp = xsp_ref[...]
    dinv = xsp[:, 4:5]
    a1 = dinv * (S[:, :4] + xsp[:, :4])
    h1 = lax.dot_general(a1, w1_ref[...], (((1,), (0,)), ((), ())),
                         preferred_element_type=f32)
    h1 = jnp.maximum(h1 + b1_ref[...], 0.0)
    hs = dinv * h1
    hsa_ref[...] = hs[:, :16]
    hsb_ref[...] = hs[:, 16:]


def _tc_layer1(s1p, xsp, W1, b1):
    return pl.pallas_call(
        _tc_layer1_body,
        grid=(_GRID,),
        in_specs=[
            pl.BlockSpec((NC, _BN, 16), lambda i: (0, i, 0)),
            pl.BlockSpec((_BN, 16), lambda i: (i, 0)),
            pl.BlockSpec((4, 32), lambda i: (0, 0)),
            pl.BlockSpec((1, 32), lambda i: (0, 0)),
        ],
        out_specs=[
            pl.BlockSpec((_BN, 16), lambda i: (i, 0)),
            pl.BlockSpec((_BN, 16), lambda i: (i, 0)),
        ],
        out_shape=[
            jax.ShapeDtypeStruct((NP, 16), f32),
            jax.ShapeDtypeStruct((NP, 16), f32),
        ],
    )(s1p, xsp, W1, b1)


def _tc_layer2_body(s2p_ref, hsa_ref, hsb_ref, xsp_ref, w2_ref, b2_ref,
                    h2_ref):
    S2 = jnp.concatenate([s2p_ref[0], s2p_ref[1]], axis=1)
    hs = jnp.concatenate([hsa_ref[...], hsb_ref[...]], axis=1)
    dinv = xsp_ref[...][:, 4:5]
    a2 = dinv * (S2 + hs)
    h2 = lax.dot_general(a2, w2_ref[...], (((1,), (0,)), ((), ())),
                         preferred_element_type=f32)
    h2_ref[...] = jnp.maximum(h2 + b2_ref[...], 0.0)


def _tc_layer2(s2p, hsa, hsb, xsp, W2, b2):
    return pl.pallas_call(
        _tc_layer2_body,
        grid=(_GRID,),
        in_specs=[
            pl.BlockSpec((NC, _BN, 16), lambda i: (0, i, 0)),
            pl.BlockSpec((_BN, 16), lambda i: (i, 0)),
            pl.BlockSpec((_BN, 16), lambda i: (i, 0)),
            pl.BlockSpec((_BN, 16), lambda i: (i, 0)),
            pl.BlockSpec((32, 32), lambda i: (0, 0)),
            pl.BlockSpec((1, 32), lambda i: (0, 0)),
        ],
        out_specs=pl.BlockSpec((_BN, 32), lambda i: (i, 0)),
        out_shape=jax.ShapeDtypeStruct((NP, 32), f32),
    )(s2p, hsa, hsb, xsp, W2, b2)


def _tc_heads_body(sums_ref, cnt_ref, wo_ref, bo_ref, wb_ref, bb_ref,
                   o_ref, b_ref):
    sums = sums_ref[0] + sums_ref[1]
    cnt = jnp.sum(cnt_ref[...], axis=0)[:, None]
    pooled = sums / jnp.maximum(cnt, 1.0)
    o = lax.dot_general(pooled, wo_ref[...], (((1,), (0,)), ((), ())),
                        preferred_element_type=f32)
    b = lax.dot_general(pooled, wb_ref[...], (((1,), (0,)), ((), ())),
                        preferred_element_type=f32)
    o_ref[...] = jax.nn.sigmoid(o + bo_ref[...])
    b_ref[...] = jax.nn.sigmoid(b + bb_ref[...])


def _tc_heads(sums, cnt, Wo, bo, Wb, bb):
    return pl.pallas_call(
        _tc_heads_body,
        out_shape=[
            jax.ShapeDtypeStruct((G, 1), f32),
            jax.ShapeDtypeStruct((G, 1), f32),
        ],
    )(sums, cnt, Wo, bo, Wb, bb)


def kernel(x, edge_index, edge_weight, batch, W1, b1, W2, b2, Wo, bo, Wb, bb):
    row2d = edge_index[0].reshape(EG, 128)
    col2d = edge_index[1].reshape(EG, 128)
    ew2d = edge_weight.reshape(EG, 128)
    batch2d = batch.reshape(_PG, _PC)

    degp = _deg(col2d, ew2d)
    xsp = _tc_xs(degp, x)
    (xsg,) = _repack_one(xsp)
    s1p = _prop_l1(xsg, xsg, row2d, col2d, ew2d)
    hsa, hsb = _tc_layer1(s1p, xsp, W1, b1.reshape(1, 32))
    hsag, hsbg = _repack_two(hsa, hsb)
    s2p = _prop_l2(hsag, hsbg, row2d, col2d, ew2d)
    h2 = _tc_layer2(s2p, hsa, hsb, xsp, W2, b2.reshape(1, 32))
    sums, cnt = _pool(h2, batch2d)
    orange, blue = _tc_heads(sums, cnt, Wo, bo.reshape(1, 1),
                             Wb, bb.reshape(1, 1))
    return (orange, blue)
